# Initial kernel scaffold; baseline (speedup 1.0000x reference)
import jax
import jax.numpy as jnp
from jax.experimental import pallas as pl


def kernel(x, table, W1, b1, W2, b2):
    raise NotImplementedError("write your pallas kernel here")


# speedup vs baseline: 19.5125x; 19.5125x over previous
import functools

import jax
import jax.numpy as jnp
from jax import lax
from jax.experimental import pallas as pl
from jax.experimental.pallas import tpu as pltpu
from jax.experimental.pallas import tpu_sc as plsc

B = 16384
L = 200
D = 128
H1 = 64
H2 = 2

NC, NS = 2, 16
NW = NC * NS
GS = 128
QN = 4
SPW = QN * GS

_mesh = plsc.VectorSubcoreMesh(
    core_axis_name="c", subcore_axis_name="s", num_cores=NC, num_subcores=NS
)


@functools.partial(
    pl.kernel,
    out_type=jax.ShapeDtypeStruct((B, D), jnp.float32),
    mesh=_mesh,
    scratch_types=[
        pltpu.VMEM((SPW, D), jnp.float32),
        pltpu.VMEM((2, QN, GS), jnp.int32),
        pltpu.SemaphoreType.DMA,
        pltpu.SemaphoreType.DMA,
    ],
)
def _emb_sum(idx_hbm, table_hbm, out_hbm, acc_v, idx_v, gsem, isem):
    wid = lax.axis_index("s") * NC + lax.axis_index("c")
    cbase = wid * L

    pltpu.sync_copy(idx_hbm.at[cbase], idx_v.at[0])
    pltpu.async_copy(idx_hbm.at[cbase + 1], idx_v.at[1], isem)
    d0 = [
        pltpu.async_copy(
            table_hbm.at[idx_v.at[0, q]], acc_v.at[pl.ds(q * GS, GS)], gsem
        )
        for q in range(QN)
    ]
    for dsc in d0:
        dsc.wait()

    def body(c, carry):
        b = lax.rem(c, 2)
        nb = lax.rem(c + 1, 2)
        pltpu.make_async_copy(idx_hbm.at[cbase + c], idx_v.at[b], isem).wait()

        @pl.when(c < L - 1)
        def _():
            pltpu.async_copy(idx_hbm.at[cbase + c + 1], idx_v.at[nb], isem)

        ds = [
            pltpu.async_copy(
                table_hbm.at[idx_v.at[b, q]],
                acc_v.at[pl.ds(q * GS, GS)],
                gsem,
                add=True,
            )
            for q in range(QN)
        ]
        for dsc in ds:
            dsc.wait()
        return carry

    lax.fori_loop(1, L, body, 0)
    pltpu.sync_copy(acc_v, out_hbm.at[pl.ds(wid * SPW, SPW)])


BLK = 2048


def _mlp_body(h_ref, w1_ref, b1_ref, w2_ref, b2_ref, o_ref):
    h = h_ref[...]
    z = jnp.dot(h, w1_ref[...], preferred_element_type=jnp.float32) + b1_ref[...]
    z = jnp.maximum(z, 0.0)
    o_ref[...] = (
        jnp.dot(z, w2_ref[...], preferred_element_type=jnp.float32) + b2_ref[...]
    )


_mlp = pl.pallas_call(
    _mlp_body,
    grid=(B // BLK,),
    in_specs=[
        pl.BlockSpec((BLK, D), lambda i: (i, 0)),
        pl.BlockSpec((D, H1), lambda i: (0, 0)),
        pl.BlockSpec((1, H1), lambda i: (0, 0)),
        pl.BlockSpec((H1, H2), lambda i: (0, 0)),
        pl.BlockSpec((1, H2), lambda i: (0, 0)),
    ],
    out_specs=pl.BlockSpec((BLK, H2), lambda i: (i, 0)),
    out_shape=jax.ShapeDtypeStruct((B, H2), jnp.float32),
)


def kernel(x, table, W1, b1, W2, b2):
    x = x.astype(jnp.int32)
    idx = x.reshape(NW, QN, GS, L).transpose(0, 3, 1, 2).reshape(NW * L, QN, GS)
    sums = _emb_sum(idx, table)
    w1t = (W1 * (1.0 / L)).T
    return _mlp(sums, w1t, b1.reshape(1, H1), W2.T, b2.reshape(1, H2))


# speedup vs baseline: 23.0734x; 1.1825x over previous
import functools

import jax
import jax.numpy as jnp
from jax import lax
from jax.experimental import pallas as pl
from jax.experimental.pallas import tpu as pltpu
from jax.experimental.pallas import tpu_sc as plsc

B = 16384
L = 200
D = 128
H1 = 64
H2 = 2

NC, NS = 2, 16
NW = NC * NS
GS = 128
HN = 2
Q2 = 2
HS = Q2 * GS
SPW = HN * HS

_mesh = plsc.VectorSubcoreMesh(
    core_axis_name="c", subcore_axis_name="s", num_cores=NC, num_subcores=NS
)


@functools.partial(
    pl.kernel,
    out_type=jax.ShapeDtypeStruct((2, B, D), jnp.float32),
    mesh=_mesh,
    scratch_types=[
        pltpu.VMEM((2, HS, D), jnp.float32),
        pltpu.VMEM((3, Q2, GS), jnp.int32),
        pltpu.SemaphoreType.DMA,
        pltpu.SemaphoreType.DMA,
    ],
)
def _emb_sum(idx_hbm, table_hbm, out_hbm, acc_v, idx_v, gsem, isem):
    wid = lax.axis_index("s") * NC + lax.axis_index("c")

    def gathers(buf, par, add):
        return [
            pltpu.async_copy(
                table_hbm.at[idx_v.at[buf, q]],
                acc_v.at[par, pl.ds(q * GS, GS)],
                gsem,
                add=add,
            )
            for q in range(Q2)
        ]

    def wait_gathers():
        for q in range(Q2):
            pltpu.make_async_copy(
                table_hbm.at[idx_v.at[0, q]],
                acc_v.at[0, pl.ds(q * GS, GS)],
                gsem,
            ).wait()

    for h in range(HN):
        hbase = (wid * HN + h) * L

        pltpu.sync_copy(idx_hbm.at[hbase], idx_v.at[0])
        pltpu.async_copy(idx_hbm.at[hbase + 1], idx_v.at[1], isem)
        pltpu.async_copy(idx_hbm.at[hbase + 2], idx_v.at[2], isem)
        gathers(0, 0, add=False)
        pltpu.make_async_copy(idx_hbm.at[hbase + 1], idx_v.at[1], isem).wait()
        gathers(1, 1, add=False)
        wait_gathers()
        pltpu.async_copy(idx_hbm.at[hbase + 3], idx_v.at[0], isem)

        def body(j, carry):
            buf = lax.rem(j, 3)
            par = lax.rem(j, 2)
            pltpu.make_async_copy(idx_hbm.at[hbase + j], idx_v.at[buf], isem).wait()
            gathers(buf, par, add=True)
            wait_gathers()

            @pl.when(j < L - 2)
            def _():
                pltpu.async_copy(
                    idx_hbm.at[hbase + j + 2], idx_v.at[lax.rem(j + 2, 3)], isem
                )

            return carry

        lax.fori_loop(2, L, body, 0)
        wait_gathers()

        obase = wid * SPW + h * HS
        pltpu.sync_copy(acc_v.at[0], out_hbm.at[0, pl.ds(obase, HS)])
        pltpu.sync_copy(acc_v.at[1], out_hbm.at[1, pl.ds(obase, HS)])


BLK = 2048


def _mlp_body(h0_ref, h1_ref, w1_ref, b1_ref, w2_ref, b2_ref, o_ref):
    h = h0_ref[0] + h1_ref[0]
    z = jnp.dot(h, w1_ref[...], preferred_element_type=jnp.float32) + b1_ref[...]
    z = jnp.maximum(z, 0.0)
    o_ref[...] = (
        jnp.dot(z, w2_ref[...], preferred_element_type=jnp.float32) + b2_ref[...]
    )


_mlp = pl.pallas_call(
    _mlp_body,
    grid=(B // BLK,),
    in_specs=[
        pl.BlockSpec((1, BLK, D), lambda i: (0, i, 0)),
        pl.BlockSpec((1, BLK, D), lambda i: (1, i, 0)),
        pl.BlockSpec((D, H1), lambda i: (0, 0)),
        pl.BlockSpec((1, H1), lambda i: (0, 0)),
        pl.BlockSpec((H1, H2), lambda i: (0, 0)),
        pl.BlockSpec((1, H2), lambda i: (0, 0)),
    ],
    out_specs=pl.BlockSpec((BLK, H2), lambda i: (i, 0)),
    out_shape=jax.ShapeDtypeStruct((B, H2), jnp.float32),
)


def kernel(x, table, W1, b1, W2, b2):
    x = x.astype(jnp.int32)
    idx = (
        x.reshape(NW, HN, Q2, GS, L)
        .transpose(0, 1, 4, 2, 3)
        .reshape(NW * HN * L, Q2, GS)
    )
    sums = _emb_sum(idx, table)
    w1t = (W1 * (1.0 / L)).T
    return _mlp(sums, sums, w1t, b1.reshape(1, H1), W2.T, b2.reshape(1, H2))
